# baseline (speedup 1.0000x reference)
import functools

import jax
import jax.numpy as jnp
from jax import lax
from jax.experimental import pallas as pl
from jax.experimental.pallas import tpu as pltpu
from jax.experimental.pallas import tpu_sc as plsc

_NUM_POIS = 1000000
_EMBED_DIM = 64
_BATCH = 4096
_HIST = 50

_NC = 2
_NS = 16
_NW = _NC * _NS
_BT = _BATCH // _NW

_FULL_UNITS = _NUM_POIS // 128
_TAIL0 = _FULL_UNITS * 128
_NU = -(-_FULL_UNITS // _NW)

_mesh = plsc.VectorSubcoreMesh(core_axis_name="c", subcore_axis_name="s")
_params = pltpu.CompilerParams(use_tc_tiling_on_sc=True,
                               needs_layout_passes=False)


@functools.partial(
    pl.kernel,
    mesh=_mesh,
    out_type=jax.ShapeDtypeStruct((_NUM_POIS // 2, 128), jnp.float32),
    scratch_types=[
        pltpu.VMEM((2, 64, 128), jnp.float32),
        pltpu.VMEM((2, 64, 128), jnp.float32),
        pltpu.VMEM((64, 64), jnp.float32),
        pltpu.SemaphoreType.DMA((2,)),
        pltpu.SemaphoreType.DMA((2,)),
        pltpu.SemaphoreType.DMA,
    ],
    compiler_params=_params,
)
def _sc_convert(tab_t, lin, in_v, stage_v, tail_v, isem, osem, tsem):
    wid = lax.axis_index("s") * _NC + lax.axis_index("c")
    lanes = lax.iota(jnp.int32, 16)
    zeros = lanes * 0
    pat_d = [lanes + d0 for d0 in range(0, _EMBED_DIM, 16)]

    def unit_col(k):
        u = wid + _NW * k
        return jnp.minimum(u, _FULL_UNITS - 1) * 128

    def fetch(k, buf):
        col0 = pl.multiple_of(unit_col(k), 128)
        pltpu.async_copy(tab_t.at[:, pl.ds(col0, 128)], in_v.at[buf],
                         isem.at[buf])

    def fetch_wait(buf):
        pltpu.make_async_copy(tab_t.at[:, pl.ds(0, 128)], in_v.at[buf],
                              isem.at[buf]).wait()

    def transpose(buf, n_pairs):
        for p in range(n_pairs):
            vs = [plsc.load_gather(in_v.at[buf],
                                   [pat_d[k], zeros + (2 * p + par)])
                  for par in (0, 1) for k in range(4)]
            for j, (par, k) in enumerate((par, k) for par in (0, 1)
                                         for k in range(4)):
                stage_v[buf, p, pl.ds(64 * par + 16 * k, 16)] = vs[j]

    def flush(k, buf):
        p0 = pl.multiple_of(unit_col(k) // 2, 64)
        pltpu.async_copy(stage_v.at[buf], lin.at[pl.ds(p0, 64), :],
                         osem.at[buf])

    def flush_wait(buf):
        pltpu.make_async_copy(stage_v.at[buf], lin.at[pl.ds(0, 64), :],
                              osem.at[buf]).wait()

    fetch(0, 0)

    def step(t, carry):
        for par in (0, 1):
            k = 2 * t + par
            fetch_wait(par)
            pl.when(k + 1 < _NU)(lambda: fetch(k + 1, 1 - par))
            pl.when(k >= 2)(lambda: flush_wait(par))
            transpose(par, 64)
            flush(k, par)
        return carry

    lax.fori_loop(0, _NU // 2, step, 0)
    fetch_wait(0)
    flush_wait(0)
    transpose(0, 64)
    flush(_NU - 1, 0)
    flush_wait(1)
    flush_wait(0)

    @pl.when(wid == 0)
    def _tail():
        for d in range(_EMBED_DIM):
            pltpu.async_copy(tab_t.at[d, pl.ds(_TAIL0, 64)], tail_v.at[d],
                             tsem)
        for d in range(_EMBED_DIM):
            pltpu.make_async_copy(tab_t.at[0, pl.ds(0, 64)], tail_v.at[d],
                                  tsem).wait()
        for p in range(32):
            vs = [plsc.load_gather(tail_v, [pat_d[k], zeros + (2 * p + par)])
                  for par in (0, 1) for k in range(4)]
            for j, (par, k) in enumerate((par, k) for par in (0, 1)
                                         for k in range(4)):
                stage_v[0, p, pl.ds(64 * par + 16 * k, 16)] = vs[j]
        pltpu.async_copy(stage_v.at[0, pl.ds(0, 32), :],
                         lin.at[pl.ds(_TAIL0 // 2, 32), :], tsem)
        pltpu.make_async_copy(stage_v.at[0, pl.ds(0, 32), :],
                              lin.at[pl.ds(0, 32), :], tsem).wait()


@functools.partial(
    pl.kernel,
    mesh=_mesh,
    out_type=jax.ShapeDtypeStruct((_HIST, _EMBED_DIM, _BATCH), jnp.float32),
    scratch_types=[
        pltpu.VMEM((4, 1, _BT), jnp.int32),
        pltpu.VMEM((4, 1, _BT), jnp.int32),
        pltpu.VMEM((4, 1, _BT), jnp.int32),
        pltpu.VMEM((4, _BT, 128), jnp.float32),
        pltpu.VMEM((2, _EMBED_DIM, _BT), jnp.float32),
        pltpu.SemaphoreType.DMA((4,)),
        pltpu.SemaphoreType.DMA((4,)),
        pltpu.SemaphoreType.DMA((2,)),
    ],
    compiler_params=_params,
)
def _sc_gather(idx_t, lin, out3, raw_v, gidx_v, par_v, rows_v, stage_v,
               isem, gsem, wsem):
    wid = lax.axis_index("s") * _NC + lax.axis_index("c")
    b0 = pl.multiple_of(wid * _BT, 128)
    lanes = lax.iota(jnp.int32, 16)
    lanes_c0 = [lanes + (16 * g) for g in range(_BT // 16)]

    def idx_fetch(h, buf):
        pltpu.async_copy(idx_t.at[h, pl.ds(b0, _BT)], raw_v.at[buf, 0],
                         isem.at[buf])

    def idx_wait(buf):
        pltpu.make_async_copy(idx_t.at[0, pl.ds(0, _BT)], raw_v.at[buf, 0],
                              isem.at[buf]).wait()

    def idx_split(buf):
        for g in range(_BT // 16):
            v = raw_v[buf, 0, pl.ds(16 * g, 16)]
            gidx_v[buf, 0, pl.ds(16 * g, 16)] = v >> 1
            par_v[buf, 0, pl.ds(16 * g, 16)] = v & 1

    def gather(buf):
        pltpu.async_copy(lin.at[gidx_v.at[buf, 0]], rows_v.at[buf],
                         gsem.at[buf])

    def gather_wait(buf):
        pltpu.make_async_copy(lin.at[gidx_v.at[buf, 0]], rows_v.at[buf],
                              gsem.at[buf]).wait()

    def transpose(buf, sbuf):
        for g in range(_BT // 16):
            vpre = par_v[buf, 0, pl.ds(16 * g, 16)] * _EMBED_DIM
            for d0 in range(0, _EMBED_DIM, 8):
                vs = [plsc.load_gather(rows_v.at[buf],
                                       [lanes_c0[g], vpre + (d0 + j)])
                      for j in range(8)]
                for j in range(8):
                    stage_v[sbuf, d0 + j, pl.ds(16 * g, 16)] = vs[j]

    def writeback(h, sbuf):
        pltpu.async_copy(stage_v.at[sbuf], out3.at[h, :, pl.ds(b0, _BT)],
                         wsem.at[sbuf])

    def writeback_wait(sbuf):
        pltpu.make_async_copy(stage_v.at[sbuf], out3.at[0, :, pl.ds(0, _BT)],
                              wsem.at[sbuf]).wait()

    for b in range(4):
        idx_fetch(b, b)
    for b in range(3):
        idx_wait(b)
        idx_split(b)
        gather(b)

    def step(t, carry):
        for q in range(4):
            h = 4 * t + q
            gather_wait(q)

            qn = (q + 3) % 4

            def _advance():
                idx_wait(qn)
                idx_split(qn)
                gather(qn)
            pl.when(h + 3 < _HIST)(_advance)
            pl.when(h + 4 < _HIST)(lambda: idx_fetch(h + 4, q))
            pl.when(h >= 2)(lambda: writeback_wait(q % 2))
            transpose(q, q % 2)
            writeback(h, q % 2)
        return carry

    lax.fori_loop(0, (_HIST - 2) // 4, step, 0)
    for h, q in ((48, 0), (49, 1)):
        gather_wait(q)
        writeback_wait(q % 2)
        transpose(q, q % 2)
        writeback(h, q % 2)
    writeback_wait(0)
    writeback_wait(1)


def kernel(poi_idx, geo_embedding_weight):
    lin = _sc_convert(geo_embedding_weight.T)
    out3 = _sc_gather(poi_idx.T.astype(jnp.int32), lin)
    return jnp.transpose(out3, (2, 0, 1))


# speedup vs baseline: 1.1320x; 1.1320x over previous
import functools

import jax
import jax.numpy as jnp
from jax import lax
from jax.experimental import pallas as pl
from jax.experimental.pallas import tpu as pltpu
from jax.experimental.pallas import tpu_sc as plsc

_NUM_POIS = 1000000
_EMBED_DIM = 64
_BATCH = 4096
_HIST = 50

_NC = 2
_NS = 16
_NW = _NC * _NS
_BT = _BATCH // _NW

_FULL_UNITS = _NUM_POIS // 128
_TAIL0 = _FULL_UNITS * 128
_NU = -(-_FULL_UNITS // _NW)

_mesh = plsc.VectorSubcoreMesh(core_axis_name="c", subcore_axis_name="s")
_params = pltpu.CompilerParams(use_tc_tiling_on_sc=True,
                               needs_layout_passes=False)


@functools.partial(
    pl.kernel,
    mesh=_mesh,
    out_type=jax.ShapeDtypeStruct((_NUM_POIS // 2, 128), jnp.float32),
    scratch_types=[
        pltpu.VMEM((2, 64, 128), jnp.float32),
        pltpu.VMEM((2, 64, 128), jnp.float32),
        pltpu.VMEM((64, 64), jnp.float32),
        pltpu.SemaphoreType.DMA((2,)),
        pltpu.SemaphoreType.DMA((2,)),
        pltpu.SemaphoreType.DMA,
    ],
    compiler_params=_params,
)
def _sc_convert(tab_t, lin, in_v, stage_v, tail_v, isem, osem, tsem):
    wid = lax.axis_index("s") * _NC + lax.axis_index("c")
    lanes = lax.iota(jnp.int32, 16)
    zeros = lanes * 0
    pat_d = [lanes + d0 for d0 in range(0, _EMBED_DIM, 16)]

    def unit_col(k):
        u = wid + _NW * k
        return jnp.minimum(u, _FULL_UNITS - 1) * 128

    def fetch(k, buf):
        col0 = pl.multiple_of(unit_col(k), 128)
        pltpu.async_copy(tab_t.at[:, pl.ds(col0, 128)], in_v.at[buf],
                         isem.at[buf])

    def fetch_wait(buf):
        pltpu.make_async_copy(tab_t.at[:, pl.ds(0, 128)], in_v.at[buf],
                              isem.at[buf]).wait()

    def transpose_from(src, buf, n_pairs):
        def pair(p, cols):
            vs = [plsc.load_gather(src, [pat_d[k], cols + par])
                  for par in (0, 1) for k in range(4)]
            for j, (par, k) in enumerate((par, k) for par in (0, 1)
                                         for k in range(4)):
                stage_v[buf, p, pl.ds(64 * par + 16 * k, 16)] = vs[j]
            return cols + 2

        def quad(t, cols):
            for u in range(4):
                cols = pair(4 * t + u, cols)
            return cols

        lax.fori_loop(0, n_pairs // 4, quad, zeros)

    def transpose(buf, n_pairs):
        transpose_from(in_v.at[buf], buf, n_pairs)

    def flush(k, buf):
        p0 = pl.multiple_of(unit_col(k) // 2, 64)
        pltpu.async_copy(stage_v.at[buf], lin.at[pl.ds(p0, 64), :],
                         osem.at[buf])

    def flush_wait(buf):
        pltpu.make_async_copy(stage_v.at[buf], lin.at[pl.ds(0, 64), :],
                              osem.at[buf]).wait()

    fetch(0, 0)

    def step(t, carry):
        for par in (0, 1):
            k = 2 * t + par
            fetch_wait(par)
            pl.when(k + 1 < _NU)(lambda: fetch(k + 1, 1 - par))
            pl.when(k >= 2)(lambda: flush_wait(par))
            transpose(par, 64)
            flush(k, par)
        return carry

    lax.fori_loop(0, _NU // 2, step, 0)
    fetch_wait(0)
    flush_wait(0)
    transpose(0, 64)
    flush(_NU - 1, 0)
    flush_wait(1)
    flush_wait(0)

    @pl.when(wid == 0)
    def _tail():
        for d in range(_EMBED_DIM):
            pltpu.async_copy(tab_t.at[d, pl.ds(_TAIL0, 64)], tail_v.at[d],
                             tsem)
        for d in range(_EMBED_DIM):
            pltpu.make_async_copy(tab_t.at[0, pl.ds(0, 64)], tail_v.at[d],
                                  tsem).wait()
        transpose_from(tail_v, 0, 32)
        pltpu.async_copy(stage_v.at[0, pl.ds(0, 32), :],
                         lin.at[pl.ds(_TAIL0 // 2, 32), :], tsem)
        pltpu.make_async_copy(stage_v.at[0, pl.ds(0, 32), :],
                              lin.at[pl.ds(0, 32), :], tsem).wait()


@functools.partial(
    pl.kernel,
    mesh=_mesh,
    out_type=jax.ShapeDtypeStruct((_HIST, _EMBED_DIM, _BATCH), jnp.float32),
    scratch_types=[
        pltpu.VMEM((4, 1, _BT), jnp.int32),
        pltpu.VMEM((4, 1, _BT), jnp.int32),
        pltpu.VMEM((4, 1, _BT), jnp.int32),
        pltpu.VMEM((4, _BT, 128), jnp.float32),
        pltpu.VMEM((2, _EMBED_DIM, _BT), jnp.float32),
        pltpu.SemaphoreType.DMA((4,)),
        pltpu.SemaphoreType.DMA((4,)),
        pltpu.SemaphoreType.DMA((2,)),
    ],
    compiler_params=_params,
)
def _sc_gather(idx_t, lin, out3, raw_v, gidx_v, par_v, rows_v, stage_v,
               isem, gsem, wsem):
    wid = lax.axis_index("s") * _NC + lax.axis_index("c")
    b0 = pl.multiple_of(wid * _BT, 128)
    lanes = lax.iota(jnp.int32, 16)
    lanes_c0 = [lanes + (16 * g) for g in range(_BT // 16)]

    def idx_fetch(h, buf):
        pltpu.async_copy(idx_t.at[h, pl.ds(b0, _BT)], raw_v.at[buf, 0],
                         isem.at[buf])

    def idx_wait(buf):
        pltpu.make_async_copy(idx_t.at[0, pl.ds(0, _BT)], raw_v.at[buf, 0],
                              isem.at[buf]).wait()

    def idx_split(buf):
        for g in range(_BT // 16):
            v = raw_v[buf, 0, pl.ds(16 * g, 16)]
            gidx_v[buf, 0, pl.ds(16 * g, 16)] = v >> 1
            par_v[buf, 0, pl.ds(16 * g, 16)] = v & 1

    def gather(buf):
        pltpu.async_copy(lin.at[gidx_v.at[buf, 0]], rows_v.at[buf],
                         gsem.at[buf])

    def gather_wait(buf):
        pltpu.make_async_copy(lin.at[gidx_v.at[buf, 0]], rows_v.at[buf],
                              gsem.at[buf]).wait()

    def transpose(buf, sbuf):
        for g in range(_BT // 16):
            vpre = par_v[buf, 0, pl.ds(16 * g, 16)] * _EMBED_DIM
            for d0 in range(0, _EMBED_DIM, 8):
                vs = [plsc.load_gather(rows_v.at[buf],
                                       [lanes_c0[g], vpre + (d0 + j)])
                      for j in range(8)]
                for j in range(8):
                    stage_v[sbuf, d0 + j, pl.ds(16 * g, 16)] = vs[j]

    def writeback(h, sbuf):
        pltpu.async_copy(stage_v.at[sbuf], out3.at[h, :, pl.ds(b0, _BT)],
                         wsem.at[sbuf])

    def writeback_wait(sbuf):
        pltpu.make_async_copy(stage_v.at[sbuf], out3.at[0, :, pl.ds(0, _BT)],
                              wsem.at[sbuf]).wait()

    for b in range(4):
        idx_fetch(b, b)
    for b in range(3):
        idx_wait(b)
        idx_split(b)
        gather(b)

    def step(t, carry):
        for q in range(4):
            h = 4 * t + q
            gather_wait(q)

            qn = (q + 3) % 4

            def _advance():
                idx_wait(qn)
                idx_split(qn)
                gather(qn)
            pl.when(h + 3 < _HIST)(_advance)
            pl.when(h + 4 < _HIST)(lambda: idx_fetch(h + 4, q))
            pl.when(h >= 2)(lambda: writeback_wait(q % 2))
            transpose(q, q % 2)
            writeback(h, q % 2)
        return carry

    lax.fori_loop(0, (_HIST - 2) // 4, step, 0)
    for h, q in ((48, 0), (49, 1)):
        gather_wait(q)
        writeback_wait(q % 2)
        transpose(q, q % 2)
        writeback(h, q % 2)
    writeback_wait(0)
    writeback_wait(1)


def kernel(poi_idx, geo_embedding_weight):
    lin = _sc_convert(geo_embedding_weight.T)
    out3 = _sc_gather(poi_idx.T.astype(jnp.int32), lin)
    return jnp.transpose(out3, (2, 0, 1))


# speedup vs baseline: 1.9028x; 1.6810x over previous
import functools

import jax
import jax.numpy as jnp
from jax import lax
from jax.experimental import pallas as pl
from jax.experimental.pallas import tpu as pltpu
from jax.experimental.pallas import tpu_sc as plsc

_NUM_POIS = 1000000
_EMBED_DIM = 64
_BATCH = 4096
_HIST = 50

_NC = 2
_NS = 16
_NW = _NC * _NS
_TOTAL = _BATCH * _HIST
_B_PER_W = _TOTAL // _NW
_CHUNK = 128
_N_CHUNKS = _B_PER_W // _CHUNK
_NBUF = 5

_mesh = plsc.VectorSubcoreMesh(core_axis_name="c", subcore_axis_name="s")


@functools.partial(
    pl.kernel,
    mesh=_mesh,
    out_type=jax.ShapeDtypeStruct((_TOTAL, _EMBED_DIM), jnp.float32),
    scratch_types=[
        pltpu.VMEM((_N_CHUNKS, _CHUNK), jnp.int32),
        pltpu.VMEM((_NBUF, _CHUNK, _EMBED_DIM), jnp.float32),
        pltpu.SemaphoreType.DMA((_NBUF,)),
        pltpu.SemaphoreType.DMA((_NBUF,)),
    ],
    compiler_params=pltpu.CompilerParams(use_tc_tiling_on_sc=False),
)
def _sc_gather(idx_hbm, table_hbm, out_hbm, idx_v, rows_v, gsem, wsem):
    wid = lax.axis_index("s") * _NC + lax.axis_index("c")
    base = wid * _B_PER_W
    pltpu.sync_copy(idx_hbm.at[wid], idx_v)

    def gather(j, b):
        pltpu.async_copy(table_hbm.at[idx_v.at[j]], rows_v.at[b], gsem.at[b])

    def writeback(j, b):
        off = pl.multiple_of(base + j * _CHUNK, _CHUNK)
        return pltpu.async_copy(rows_v.at[b], out_hbm.at[pl.ds(off, _CHUNK)],
                                wsem.at[b])

    for b in range(_NBUF):
        gather(b, b)

    def step(i, carry):
        g = i * _NBUF
        for b in range(_NBUF):
            j = g + b
            pltpu.make_async_copy(table_hbm.at[idx_v.at[j]], rows_v.at[b],
                                  gsem.at[b]).wait()
            writeback(j, b).wait()
            gather(j + _NBUF, b)
        return carry

    lax.fori_loop(0, _N_CHUNKS // _NBUF - 1, step, 0)

    handles = []
    for b in range(_NBUF):
        j = _N_CHUNKS - _NBUF + b
        pltpu.make_async_copy(table_hbm.at[idx_v.at[j]], rows_v.at[b],
                              gsem.at[b]).wait()
        handles.append(writeback(j, b))
    for h in handles:
        h.wait()


def kernel(poi_idx, geo_embedding_weight):
    idx = poi_idx.astype(jnp.int32).reshape(_NW, _N_CHUNKS, _CHUNK)
    out = _sc_gather(idx, geo_embedding_weight)
    return out.reshape(_BATCH, _HIST, _EMBED_DIM)
